# baseline (speedup 1.0000x reference)
import functools

import jax
import jax.numpy as jnp
from jax import lax
from jax.experimental import pallas as pl
from jax.experimental.pallas import tpu as pltpu
from jax.experimental.pallas import tpu_sc as plsc

_VOCAB = 100000
_EMB = 32
_HID = 128
_CTX = 3
_BATCH = 1024
_NT = 2048
_GW = 4 * _EMB


def _gather_sc(emb4, idx_flat):
    info = plsc.get_sparse_core_info()
    nc, ns = info.num_cores, info.num_subcores
    nw = nc * ns
    n = idx_flat.shape[0]
    per = n // nw
    mesh = plsc.VectorSubcoreMesh(core_axis_name="c", subcore_axis_name="s")

    @functools.partial(
        pl.kernel,
        mesh=mesh,
        out_type=jax.ShapeDtypeStruct((n, _GW), jnp.float32),
        scratch_types=[
            pltpu.VMEM((per,), jnp.int32),
            pltpu.VMEM((per,), jnp.int32),
            pltpu.VMEM((per, _GW), jnp.float32),
            pltpu.SemaphoreType.DMA,
        ],
    )
    def gather_k(table_hbm, idx_hbm, out_hbm, idx_v, row_v, rows_v, sem):
        wid = lax.axis_index("s") * nc + lax.axis_index("c")
        base = wid * per
        pltpu.sync_copy(idx_hbm.at[pl.ds(base, per)], idx_v)
        for k in range(per // 16):
            sl = pl.ds(k * 16, 16)
            row_v[sl] = lax.shift_right_logical(idx_v[sl], 2)
        pltpu.async_copy(table_hbm.at[row_v], rows_v, sem).wait()
        pltpu.sync_copy(rows_v, out_hbm.at[pl.ds(base, per)])

    return gather_k(emb4, idx_flat)


def _mlp_body(x_ref, big_ref, w1r_ref, b1_ref, w2_ref, b2_ref, out_ref,
              hid_ref):
    @pl.when(pl.program_id(0) == 0)
    def _():
        xm = x_ref[...] % 4
        xm_b = jnp.concatenate(
            [jnp.broadcast_to(xm[:, c:c + 1], (_BATCH, _GW))
             for c in range(_CTX)], axis=1)
        li = lax.broadcasted_iota(jnp.int32, (_BATCH, _CTX * _GW), 1)
        live = xm_b == (li // _EMB) % 4
        bigm = jnp.where(live, big_ref[...], 0.0)
        h = jnp.dot(bigm, w1r_ref[...], preferred_element_type=jnp.float32)
        h = jnp.maximum(h + b1_ref[...], 0.0)
        hid_ref[...] = h.astype(jnp.bfloat16)

    acc = jnp.dot(hid_ref[...], w2_ref[...].astype(jnp.bfloat16),
                  preferred_element_type=jnp.float32)
    out_ref[...] = acc + b2_ref[...]


def _mlp_tc(x, big, w1r, b1, w2, b2):
    grid = pl.cdiv(_VOCAB, _NT)
    return pl.pallas_call(
        _mlp_body,
        grid=(grid,),
        in_specs=[
            pl.BlockSpec((_BATCH, _CTX), lambda i: (0, 0)),
            pl.BlockSpec((_BATCH, _CTX * _GW), lambda i: (0, 0)),
            pl.BlockSpec((_CTX * _GW, _HID), lambda i: (0, 0)),
            pl.BlockSpec((1, _HID), lambda i: (0, 0)),
            pl.BlockSpec((_HID, _NT), lambda i: (0, i)),
            pl.BlockSpec((1, _NT), lambda i: (0, i)),
        ],
        out_specs=pl.BlockSpec((_BATCH, _NT), lambda i: (0, i)),
        out_shape=jax.ShapeDtypeStruct((_BATCH, _VOCAB), jnp.float32),
        scratch_shapes=[pltpu.VMEM((_BATCH, _HID), jnp.bfloat16)],
        compiler_params=pltpu.CompilerParams(
            dimension_semantics=("arbitrary",),
        ),
    )(x, big, w1r, b1, w2, b2)


def kernel(x, emb, W1, b1, W2, b2):
    x = x.astype(jnp.int32)
    idx = x.reshape(-1)
    emb4 = emb.reshape(_VOCAB // 4, _GW)
    big = _gather_sc(emb4, idx).reshape(_BATCH, _CTX * _GW)
    w1r = jnp.broadcast_to(
        W1.reshape(_CTX, 1, _EMB, _HID),
        (_CTX, 4, _EMB, _HID)).reshape(_CTX * _GW, _HID)
    return _mlp_tc(x, big, w1r, b1.reshape(1, -1), W2, b2.reshape(1, -1))
